# baseline (speedup 1.0000x reference)
import functools

import jax
import jax.numpy as jnp
from jax.experimental import pallas as pl
from jax.experimental.pallas import tpu as pltpu

_D = 64
_MAXQ = 40
_R = 1024


def _tc_body(idx_ref, tab_ref, out_ref):
    idx = jnp.clip(idx_ref[...], 0, _MAXQ)
    classes = jax.lax.broadcasted_iota(jnp.int32, (_R, 128), 1)
    onehot = (idx == classes).astype(jnp.float32)
    out_ref[...] = jnp.dot(
        onehot, tab_ref[...], preferred_element_type=jnp.float32
    )


@functools.cache
def _make_tc(n_idx: int):
    grid = (n_idx // _R,)
    return pl.pallas_call(
        _tc_body,
        grid=grid,
        in_specs=[
            pl.BlockSpec((_R, 1), lambda i: (i, 0)),
            pl.BlockSpec((128, _D), lambda i: (0, 0)),
        ],
        out_specs=pl.BlockSpec((_R, _D), lambda i: (i, 0)),
        out_shape=jax.ShapeDtypeStruct((n_idx, _D), jnp.float32),
    )


def kernel(inputs, table):
    b, s = inputs.shape
    idx = inputs.reshape(-1, 1).astype(jnp.int32)
    tab = jnp.zeros((128, _D), jnp.float32).at[: table.shape[0]].set(table)
    out = _make_tc(idx.shape[0])(idx, tab)
    return out.reshape(b, s, _D)
